# baseline (speedup 1.0000x reference)
import functools

import jax
import jax.numpy as jnp
from jax import lax
from jax.experimental import pallas as pl
from jax.experimental.pallas import tpu as pltpu
from jax.experimental.pallas import tpu_sc as plsc

_CHUNK = 128


@functools.lru_cache(maxsize=None)
def _build(n_total, n_rows, dim, chunk):
    info = plsc.get_sparse_core_info()
    nc, ns = info.num_cores, info.num_subcores
    nw = nc * ns
    assert n_total % (nw * chunk) == 0
    b_per_w = n_total // nw
    n_steps = b_per_w // chunk

    mesh = plsc.VectorSubcoreMesh(core_axis_name="c", subcore_axis_name="s")

    @functools.partial(
        pl.kernel,
        out_type=jax.ShapeDtypeStruct((n_total, dim), jnp.float32),
        mesh=mesh,
        scratch_types=[
            pltpu.VMEM((chunk,), jnp.int32),
            pltpu.VMEM((chunk, dim), jnp.float32),
            pltpu.SemaphoreType.DMA,
        ],
        compiler_params=pltpu.CompilerParams(use_tc_tiling_on_sc=False),
    )
    def gather_kernel(idx_hbm, table_hbm, out_hbm, idx_v, rows_v, sem):
        wid = lax.axis_index("s") * nc + lax.axis_index("c")
        base = wid * b_per_w

        def step(i, carry):
            off = base + i * chunk
            pltpu.sync_copy(idx_hbm.at[pl.ds(off, chunk)], idx_v)
            pltpu.async_copy(table_hbm.at[idx_v], rows_v, sem).wait()
            pltpu.sync_copy(rows_v, out_hbm.at[pl.ds(off, chunk)])
            return carry

        lax.fori_loop(0, n_steps, step, 0)

    return gather_kernel


def kernel(indices, table):
    n_total = indices.shape[0] * indices.shape[1]
    dim = table.shape[1]
    idx = indices.reshape(n_total).astype(jnp.int32)
    out = _build(n_total, table.shape[0], dim, _CHUNK)(idx, table)
    return out.reshape(indices.shape + (dim,))


# speedup vs baseline: 1.1934x; 1.1934x over previous
import functools

import jax
import jax.numpy as jnp
from jax import lax
from jax.experimental import pallas as pl
from jax.experimental.pallas import tpu as pltpu
from jax.experimental.pallas import tpu_sc as plsc

_CHUNK = 256
_NBUF = 4


@functools.lru_cache(maxsize=None)
def _build(n_total, n_rows, dim, chunk, nbuf):
    info = plsc.get_sparse_core_info()
    nc, ns = info.num_cores, info.num_subcores
    nw = nc * ns
    assert n_total % (nw * chunk) == 0
    b_per_w = n_total // nw
    n_steps = b_per_w // chunk
    assert n_steps > nbuf

    mesh = plsc.VectorSubcoreMesh(core_axis_name="c", subcore_axis_name="s")

    @functools.partial(
        pl.kernel,
        out_type=jax.ShapeDtypeStruct((n_total, dim), jnp.float32),
        mesh=mesh,
        scratch_types=[
            pltpu.VMEM((b_per_w,), jnp.int32),
            pltpu.VMEM((nbuf, chunk, dim), jnp.float32),
            pltpu.SemaphoreType.DMA((nbuf,)),
            pltpu.SemaphoreType.DMA((nbuf,)),
        ],
        compiler_params=pltpu.CompilerParams(use_tc_tiling_on_sc=False),
    )
    def gather_kernel(idx_hbm, table_hbm, out_hbm, idx_v, rows_v, gsem, ssem):
        wid = lax.axis_index("s") * nc + lax.axis_index("c")
        base = wid * b_per_w
        pltpu.sync_copy(idx_hbm.at[pl.ds(base, b_per_w)], idx_v)

        def start_gather(j, b):
            pltpu.async_copy(
                table_hbm.at[idx_v.at[pl.ds(j * chunk, chunk)]],
                rows_v.at[b], gsem.at[b])

        def wait_gather(b):
            pltpu.make_async_copy(
                table_hbm.at[idx_v.at[pl.ds(0, chunk)]],
                rows_v.at[b], gsem.at[b]).wait()

        def start_store(i, b):
            pltpu.async_copy(
                rows_v.at[b], out_hbm.at[pl.ds(base + i * chunk, chunk)],
                ssem.at[b])

        def wait_store(b):
            pltpu.make_async_copy(
                rows_v.at[b], out_hbm.at[pl.ds(base, chunk)],
                ssem.at[b]).wait()

        for b in range(nbuf - 1):
            start_gather(b, b)
        wait_gather(0)
        start_store(0, 0)
        start_gather(nbuf - 1, nbuf - 1)

        def body(i, carry):
            b = lax.rem(i, nbuf)
            wait_gather(b)
            start_store(i, b)
            bj = lax.rem(i - 1, nbuf)
            wait_store(bj)
            start_gather(i + nbuf - 1, bj)
            return carry

        lax.fori_loop(1, n_steps - nbuf + 1, body, 0)

        for i in range(n_steps - nbuf + 1, n_steps):
            wait_gather(i % nbuf)
            start_store(i, i % nbuf)
            wait_store((i - 1) % nbuf)
        wait_store((n_steps - 1) % nbuf)

    return gather_kernel


def kernel(indices, table):
    n_total = indices.shape[0] * indices.shape[1]
    dim = table.shape[1]
    idx = indices.reshape(n_total).astype(jnp.int32)
    out = _build(n_total, table.shape[0], dim, _CHUNK, _NBUF)(idx, table)
    return out.reshape(indices.shape + (dim,))
